# baseline (speedup 1.0000x reference)
import functools

import jax
import jax.numpy as jnp
from jax import lax
from jax.experimental import pallas as pl
from jax.experimental.pallas import tpu as pltpu
from jax.experimental.pallas import tpu_sc as plsc

H = 2048
INNER = 1024
E = 8
T = 2048
B = 256
NB = 24
NPAD = NB * B

NC = 2
NS = 16
NW = NC * NS
TPW = T // NW
RCH = 16
DCH = TPW // RCH
CCH = 8
KCH = TPW // CCH


def _router_bin_kernel(x_ref, gate_w_ref, gate_b_ref,
                       dest0_ref, dest1_ref, w0_ref, bexp_ref):
    logitsT = jax.lax.dot_general(
        gate_w_ref[...], x_ref[...], (((1,), (1,)), ((), ())),
        preferred_element_type=jnp.float32,
    ) + gate_b_ref[...]
    row = jax.lax.broadcasted_iota(jnp.int32, (E, T), 0)
    m1 = jnp.max(logitsT, axis=0, keepdims=True)
    i1 = jnp.min(jnp.where(logitsT == m1, row, E), axis=0, keepdims=True)
    oh1 = (row == i1)
    masked = jnp.where(oh1, -jnp.inf, logitsT)
    m2 = jnp.max(masked, axis=0, keepdims=True)
    i2 = jnp.min(jnp.where(masked == m2, row, E), axis=0, keepdims=True)
    oh2 = (row == i2)
    diff_t = jnp.transpose(m1 - m2)
    a1c = jax.nn.sigmoid(diff_t)
    w0_ref[0] = jnp.broadcast_to(a1c, (T, 128))
    w0_ref[1] = jnp.broadcast_to(1.0 - a1c, (T, 128))

    ones_col = jnp.ones((T, 1), jnp.float32)
    cnt0 = jax.lax.dot_general(oh1.astype(jnp.float32), ones_col,
                               (((1,), (0,)), ((), ())),
                               preferred_element_type=jnp.float32)
    cnt1 = jax.lax.dot_general(oh2.astype(jnp.float32), ones_col,
                               (((1,), (0,)), ((), ())),
                               preferred_element_type=jnp.float32)
    cnt = cnt0 + cnt1

    def to16x128(v):
        return jnp.concatenate(
            [v[:, q * 128:(q + 1) * 128] for q in range(16)], axis=0)
    ID0 = to16x128(i1)
    ID1 = to16x128(i2)

    pp = jax.lax.broadcasted_iota(jnp.int32, (128, 128), 0)
    qq = jax.lax.broadcasted_iota(jnp.int32, (128, 128), 1)
    U128s = (pp < qq).astype(jnp.float32)
    aa = jax.lax.broadcasted_iota(jnp.int32, (16, 16), 0)
    bb = jax.lax.broadcasted_iota(jnp.int32, (16, 16), 1)
    L16s = (bb < aa).astype(jnp.float32)
    ones128 = jnp.ones((128, 1), jnp.float32)

    nb_e, ps_e = [], []
    run = 0.0
    for e in range(E):
        c = cnt[e, 0]
        nbe = jnp.floor((c + (B - 1)) / B)
        ps_e.append(run * B)
        nb_e.append(nbe)
        run = run + nbe
    total_blocks = run

    def dest_for(IDk, extra):
        dest = jnp.zeros((16, 128), jnp.float32)
        for e in range(E):
            Ake = (IDk == e).astype(jnp.float32)
            colpre = jax.lax.dot_general(Ake, U128s, (((1,), (0,)), ((), ())),
                                         preferred_element_type=jnp.float32)
            rowsum = jax.lax.dot_general(Ake, ones128, (((1,), (0,)), ((), ())),
                                         preferred_element_type=jnp.float32)
            rowpre = jax.lax.dot_general(L16s, rowsum, (((0,), (0,)), ((), ())),
                                         preferred_element_type=jnp.float32)
            cum = colpre + rowpre
            dest = dest + Ake * (cum + ps_e[e] + extra[e])
        return dest.astype(jnp.int32)

    dest0_ref[...] = dest_for(ID0, [0.0] * E)
    dest1_ref[...] = dest_for(ID1, [cnt0[e, 0] for e in range(E)])

    lane = jax.lax.broadcasted_iota(jnp.int32, (1, 32), 1).astype(jnp.float32)
    lane_c = jnp.minimum(lane, total_blocks - 1.0)
    acc = jnp.zeros((1, 32), jnp.float32)
    run2 = 0.0
    for e in range(E):
        run2 = run2 + nb_e[e]
        acc = acc + (lane_c >= run2).astype(jnp.float32)
    bexp = jnp.where(lane == NB, total_blocks, acc)
    bexp_ref[...] = bexp.astype(jnp.int32)


def _routing(x, gate_w, gate_b):
    return pl.pallas_call(
        _router_bin_kernel,
        out_shape=(
            jax.ShapeDtypeStruct((16, 128), jnp.int32),
            jax.ShapeDtypeStruct((16, 128), jnp.int32),
            jax.ShapeDtypeStruct((2, T, 128), jnp.float32),
            jax.ShapeDtypeStruct((1, 32), jnp.int32),
        ),
    )(x, gate_w, gate_b.reshape(E, 1))


def _dispatch_kernel(x_hbm, d_hbm, w_hbm, xs_hbm, ws_hbm,
                     idx_v, rows_v, wsp_v, sin, sout, swem):
    wid = lax.axis_index("s") * NC + lax.axis_index("c")
    toff = wid * TPW

    pltpu.sync_copy(d_hbm.at[0, wid], idx_v.at[0])
    pltpu.sync_copy(d_hbm.at[1, wid], idx_v.at[1])

    def start_in(c):
        return pltpu.async_copy(
            x_hbm.at[pl.ds(toff + c * RCH, RCH)], rows_v.at[c % 2],
            sin.at[c % 2])

    def start_out(c, k):
        return pltpu.async_copy(
            rows_v.at[c % 2], xs_hbm.at[idx_v.at[k, c]], sout.at[c % 2])

    cp_in = {0: start_in(0), 1: start_in(1)}
    cp_out = {}
    for c in range(DCH):
        cp_in[c].wait()
        cp_out[c] = (start_out(c, 0), start_out(c, 1))
        if c + 2 < DCH:
            for cp in cp_out[c]:
                cp.wait()
            cp_in[c + 2] = start_in(c + 2)
    for c in (DCH - 2, DCH - 1):
        for cp in cp_out[c]:
            cp.wait()

    for k in range(2):
        for c in range(DCH):
            pltpu.sync_copy(w_hbm.at[k, pl.ds(toff + c * RCH, RCH)], wsp_v)
            pltpu.async_copy(wsp_v, ws_hbm.at[idx_v.at[k, c]], swem).wait()


def _dispatch(x, d2, w2):
    mesh = plsc.VectorSubcoreMesh(core_axis_name="c", subcore_axis_name="s")
    kfn = pl.kernel(
        _dispatch_kernel,
        mesh=mesh,
        out_type=(
            jax.ShapeDtypeStruct((NPAD, H), jnp.float32),
            jax.ShapeDtypeStruct((NPAD, 128), jnp.float32),
        ),
        scratch_types=[
            pltpu.VMEM((2, DCH, RCH), jnp.int32),
            pltpu.VMEM((2, RCH, H), jnp.float32),
            pltpu.VMEM((RCH, 128), jnp.float32),
            pltpu.SemaphoreType.DMA((2,)),
            pltpu.SemaphoreType.DMA((2,)),
            pltpu.SemaphoreType.DMA,
        ],
    )
    return kfn(x, d2, w2)


def _wprep_kernel(fc1_ref, fc2_ref, w1_ref, w2_ref):
    w1_ref[...] = fc1_ref[...].astype(jnp.bfloat16)
    w2_ref[...] = fc2_ref[...].astype(jnp.bfloat16)


def _wprep(fc1_w, fc2_w):
    return pl.pallas_call(
        _wprep_kernel,
        grid=(E, 4),
        in_specs=[
            pl.BlockSpec((1, 512, H), lambda e, i: (e, i, 0)),
            pl.BlockSpec((1, 512, INNER), lambda e, i: (e, i, 0)),
        ],
        out_specs=[
            pl.BlockSpec((1, 512, H), lambda e, i: (e, i, 0)),
            pl.BlockSpec((1, 512, INNER), lambda e, i: (e, i, 0)),
        ],
        out_shape=(
            jax.ShapeDtypeStruct((E, 2 * INNER, H), jnp.bfloat16),
            jax.ShapeDtypeStruct((E, H, INNER), jnp.bfloat16),
        ),
        compiler_params=pltpu.CompilerParams(
            dimension_semantics=("arbitrary", "arbitrary")),
    )(fc1_w, fc2_w)


def _experts_kernel(s_ref, xs_ref, w1_ref, w2_ref, ws_ref, y_ref):
    b = pl.program_id(0)
    active = b < s_ref[NB]

    @pl.when(active)
    def _compute():
        xb = xs_ref[...].astype(jnp.bfloat16)
        h = jax.lax.dot_general(xb, w1_ref[0, :INNER], (((1,), (1,)), ((), ())),
                                preferred_element_type=jnp.float32)
        g = jax.lax.dot_general(xb, w1_ref[0, INNER:], (((1,), (1,)), ((), ())),
                                preferred_element_type=jnp.float32)
        act = 0.5 * h * (1.0 + jax.lax.erf(h * 0.7071067811865476)) * (g + 1.0)
        y = jax.lax.dot_general(act.astype(jnp.bfloat16), w2_ref[0],
                                (((1,), (1,)), ((), ())),
                                preferred_element_type=jnp.float32)
        y_ref[...] = y * ws_ref[:, 0:1]


def _experts(bexp, x_sorted, w1b, w2b, w_sorted):
    return pl.pallas_call(
        _experts_kernel,
        grid_spec=pltpu.PrefetchScalarGridSpec(
            num_scalar_prefetch=1,
            grid=(NB,),
            in_specs=[
                pl.BlockSpec((B, H), lambda b, s: (b, 0)),
                pl.BlockSpec((1, 2 * INNER, H), lambda b, s: (s[b], 0, 0)),
                pl.BlockSpec((1, H, INNER), lambda b, s: (s[b], 0, 0)),
                pl.BlockSpec((B, 128), lambda b, s: (b, 0)),
            ],
            out_specs=pl.BlockSpec((B, H), lambda b, s: (b, 0)),
        ),
        out_shape=jax.ShapeDtypeStruct((NPAD, H), jnp.float32),
        compiler_params=pltpu.CompilerParams(
            dimension_semantics=("arbitrary",)),
    )(bexp, x_sorted, w1b, w2b, w_sorted)


def _combine_kernel(y_hbm, d0_hbm, d1_hbm, out_hbm,
                    idx0_v, idx1_v, y0_v, y1_v, o_v, s0, s1, so):
    wid = lax.axis_index("s") * NC + lax.axis_index("c")
    base = wid * TPW

    pltpu.sync_copy(d0_hbm.at[wid], idx0_v)
    pltpu.sync_copy(d1_hbm.at[wid], idx1_v)

    def start_in(c):
        b = c % 2
        return (pltpu.async_copy(y_hbm.at[idx0_v.at[c]], y0_v.at[b], s0.at[b]),
                pltpu.async_copy(y_hbm.at[idx1_v.at[c]], y1_v.at[b], s1.at[b]))

    cps = {0: start_in(0), 1: start_in(1)}
    cpo = {}
    for c in range(KCH):
        b = c % 2
        for cp in cps[c]:
            cp.wait()
        if c >= 2:
            cpo[c - 2].wait()

        def body(i, carry):
            def inner(j, carry2):
                o_v[b, i, pl.ds(j * 16, 16)] = (
                    y0_v[b, i, pl.ds(j * 16, 16)]
                    + y1_v[b, i, pl.ds(j * 16, 16)])
                return carry2
            lax.fori_loop(0, H // 16, inner, 0)
            return carry
        lax.fori_loop(0, CCH, body, 0)

        cpo[c] = pltpu.async_copy(
            o_v.at[b], out_hbm.at[pl.ds(base + c * CCH, CCH)], so.at[b])
        if c + 2 < KCH:
            cps[c + 2] = start_in(c + 2)
    cpo[KCH - 2].wait()
    cpo[KCH - 1].wait()


def _combine(y_sorted, d0r, d1r):
    mesh = plsc.VectorSubcoreMesh(core_axis_name="c", subcore_axis_name="s")
    kfn = pl.kernel(
        _combine_kernel,
        mesh=mesh,
        out_type=jax.ShapeDtypeStruct((T, H), jnp.float32),
        scratch_types=[
            pltpu.VMEM((KCH, CCH), jnp.int32),
            pltpu.VMEM((KCH, CCH), jnp.int32),
            pltpu.VMEM((2, CCH, H), jnp.float32),
            pltpu.VMEM((2, CCH, H), jnp.float32),
            pltpu.VMEM((2, CCH, H), jnp.float32),
            pltpu.SemaphoreType.DMA((2,)),
            pltpu.SemaphoreType.DMA((2,)),
            pltpu.SemaphoreType.DMA((2,)),
        ],
    )
    return kfn(y_sorted, d0r, d1r)


@jax.jit
def kernel(x, gate_w, gate_b, fc1_w, fc2_w):
    dest0, dest1, w0, bexp = _routing(x, gate_w, gate_b)

    d2 = jnp.stack([dest0, dest1]).reshape(2, NW, DCH, RCH)
    x_sorted, w_sorted = _dispatch(x, d2, w0)

    w1b, w2b = _wprep(fc1_w, fc2_w)
    y = _experts(bexp.reshape(-1), x_sorted, w1b, w2b, w_sorted)

    d0r = dest0.reshape(NW, KCH, CCH)
    d1r = dest1.reshape(NW, KCH, CCH)
    return _combine(y, d0r, d1r)


# speedup vs baseline: 1.0879x; 1.0879x over previous
import functools

import jax
import jax.numpy as jnp
from jax import lax
from jax.experimental import pallas as pl
from jax.experimental.pallas import tpu as pltpu
from jax.experimental.pallas import tpu_sc as plsc

H = 2048
INNER = 1024
E = 8
T = 2048
B = 256
NB = 24
NPAD = NB * B

NC = 2
NS = 16
NW = NC * NS
TPW = T // NW
RCH = 16
DCH = TPW // RCH
CCH = 8
KCH = TPW // CCH


def _router_bin_kernel(x_ref, gate_w_ref, gate_b_ref,
                       dest0_ref, dest1_ref, w0_ref, bexp_ref):
    logitsT = jax.lax.dot_general(
        gate_w_ref[...], x_ref[...], (((1,), (1,)), ((), ())),
        preferred_element_type=jnp.float32,
    ) + gate_b_ref[...]
    row = jax.lax.broadcasted_iota(jnp.int32, (E, T), 0)
    m1 = jnp.max(logitsT, axis=0, keepdims=True)
    i1 = jnp.min(jnp.where(logitsT == m1, row, E), axis=0, keepdims=True)
    oh1 = (row == i1)
    masked = jnp.where(oh1, -jnp.inf, logitsT)
    m2 = jnp.max(masked, axis=0, keepdims=True)
    i2 = jnp.min(jnp.where(masked == m2, row, E), axis=0, keepdims=True)
    oh2 = (row == i2)
    diff_t = jnp.transpose(m1 - m2)
    a1c = jax.nn.sigmoid(diff_t)
    w0_ref[0] = jnp.broadcast_to(a1c, (T, 128))
    w0_ref[1] = jnp.broadcast_to(1.0 - a1c, (T, 128))

    ones_col = jnp.ones((T, 1), jnp.float32)
    cnt0 = jax.lax.dot_general(oh1.astype(jnp.float32), ones_col,
                               (((1,), (0,)), ((), ())),
                               preferred_element_type=jnp.float32)
    cnt1 = jax.lax.dot_general(oh2.astype(jnp.float32), ones_col,
                               (((1,), (0,)), ((), ())),
                               preferred_element_type=jnp.float32)
    cnt = cnt0 + cnt1

    def to16x128(v):
        return jnp.concatenate(
            [v[:, q * 128:(q + 1) * 128] for q in range(16)], axis=0)
    ID0 = to16x128(i1)
    ID1 = to16x128(i2)

    pp = jax.lax.broadcasted_iota(jnp.int32, (128, 128), 0)
    qq = jax.lax.broadcasted_iota(jnp.int32, (128, 128), 1)
    U128s = (pp < qq).astype(jnp.float32)
    aa = jax.lax.broadcasted_iota(jnp.int32, (16, 16), 0)
    bb = jax.lax.broadcasted_iota(jnp.int32, (16, 16), 1)
    L16s = (bb < aa).astype(jnp.float32)
    ones128 = jnp.ones((128, 1), jnp.float32)

    nb_e, ps_e = [], []
    run = 0.0
    for e in range(E):
        c = cnt[e, 0]
        nbe = jnp.floor((c + (B - 1)) / B)
        ps_e.append(run * B)
        nb_e.append(nbe)
        run = run + nbe
    total_blocks = run

    def dest_for(IDk, extra):
        dest = jnp.zeros((16, 128), jnp.float32)
        for e in range(E):
            Ake = (IDk == e).astype(jnp.float32)
            colpre = jax.lax.dot_general(Ake, U128s, (((1,), (0,)), ((), ())),
                                         preferred_element_type=jnp.float32)
            rowsum = jax.lax.dot_general(Ake, ones128, (((1,), (0,)), ((), ())),
                                         preferred_element_type=jnp.float32)
            rowpre = jax.lax.dot_general(L16s, rowsum, (((0,), (0,)), ((), ())),
                                         preferred_element_type=jnp.float32)
            cum = colpre + rowpre
            dest = dest + Ake * (cum + ps_e[e] + extra[e])
        return dest.astype(jnp.int32)

    dest0_ref[...] = dest_for(ID0, [0.0] * E)
    dest1_ref[...] = dest_for(ID1, [cnt0[e, 0] for e in range(E)])

    lane = jax.lax.broadcasted_iota(jnp.int32, (1, 32), 1).astype(jnp.float32)
    lane_c = jnp.minimum(lane, total_blocks - 1.0)
    acc = jnp.zeros((1, 32), jnp.float32)
    run2 = 0.0
    for e in range(E):
        run2 = run2 + nb_e[e]
        acc = acc + (lane_c >= run2).astype(jnp.float32)
    bexp = jnp.where(lane == NB, total_blocks, acc)
    bexp_ref[...] = bexp.astype(jnp.int32)


def _routing(x, gate_w, gate_b):
    return pl.pallas_call(
        _router_bin_kernel,
        out_shape=(
            jax.ShapeDtypeStruct((16, 128), jnp.int32),
            jax.ShapeDtypeStruct((16, 128), jnp.int32),
            jax.ShapeDtypeStruct((2, T, 128), jnp.float32),
            jax.ShapeDtypeStruct((1, 32), jnp.int32),
        ),
    )(x, gate_w, gate_b.reshape(E, 1))


def _dispatch_kernel(x_hbm, d_hbm, w_hbm, xs_hbm, ws_hbm,
                     idx_v, rows_v, wsp_v, sin, sout, swem):
    wid = lax.axis_index("s") * NC + lax.axis_index("c")
    toff = wid * TPW

    pltpu.sync_copy(d_hbm.at[0, wid], idx_v.at[0])
    pltpu.sync_copy(d_hbm.at[1, wid], idx_v.at[1])

    def start_in(c):
        return pltpu.async_copy(
            x_hbm.at[pl.ds(toff + c * RCH, RCH)], rows_v.at[c % 2],
            sin.at[c % 2])

    def start_out(c, k):
        return pltpu.async_copy(
            rows_v.at[c % 2], xs_hbm.at[idx_v.at[k, c]], sout.at[c % 2])

    cp_in = {0: start_in(0), 1: start_in(1)}
    cp_out = {}
    for c in range(DCH):
        cp_in[c].wait()
        cp_out[c] = (start_out(c, 0), start_out(c, 1))
        if c + 2 < DCH:
            for cp in cp_out[c]:
                cp.wait()
            cp_in[c + 2] = start_in(c + 2)
    for c in (DCH - 2, DCH - 1):
        for cp in cp_out[c]:
            cp.wait()

    for k in range(2):
        for c in range(DCH):
            pltpu.sync_copy(w_hbm.at[k, pl.ds(toff + c * RCH, RCH)], wsp_v)
            pltpu.async_copy(wsp_v, ws_hbm.at[idx_v.at[k, c]], swem).wait()


def _dispatch(x, d2, w2):
    mesh = plsc.VectorSubcoreMesh(core_axis_name="c", subcore_axis_name="s")
    kfn = pl.kernel(
        _dispatch_kernel,
        mesh=mesh,
        out_type=(
            jax.ShapeDtypeStruct((NPAD, H), jnp.float32),
            jax.ShapeDtypeStruct((NPAD, 128), jnp.float32),
        ),
        scratch_types=[
            pltpu.VMEM((2, DCH, RCH), jnp.int32),
            pltpu.VMEM((2, RCH, H), jnp.float32),
            pltpu.VMEM((RCH, 128), jnp.float32),
            pltpu.SemaphoreType.DMA((2,)),
            pltpu.SemaphoreType.DMA((2,)),
            pltpu.SemaphoreType.DMA,
        ],
    )
    return kfn(x, d2, w2)


def _c1_kernel(s_ref, xs_ref, fc1_ref, act_ref, w1bf_ref):
    b = pl.program_id(0)
    active = b < s_ref[NB]
    prev = s_ref[jnp.maximum(b - 1, 0)]
    change = jnp.logical_or(b == 0, s_ref[b] != prev)

    @pl.when(jnp.logical_and(active, change))
    def _load():
        w1bf_ref[...] = fc1_ref[0].astype(jnp.bfloat16)

    @pl.when(active)
    def _compute():
        xb = xs_ref[...].astype(jnp.bfloat16)
        h = jax.lax.dot_general(xb, w1bf_ref[:INNER], (((1,), (1,)), ((), ())),
                                preferred_element_type=jnp.float32)
        g = jax.lax.dot_general(xb, w1bf_ref[INNER:], (((1,), (1,)), ((), ())),
                                preferred_element_type=jnp.float32)
        act = 0.5 * h * (1.0 + jax.lax.erf(h * 0.7071067811865476)) * (g + 1.0)
        act_ref[...] = act.astype(jnp.bfloat16)


def _c1(bexp, x_sorted, fc1_w):
    return pl.pallas_call(
        _c1_kernel,
        grid_spec=pltpu.PrefetchScalarGridSpec(
            num_scalar_prefetch=1,
            grid=(NB,),
            in_specs=[
                pl.BlockSpec((B, H), lambda b, s: (b, 0)),
                pl.BlockSpec((1, 2 * INNER, H), lambda b, s: (s[b], 0, 0)),
            ],
            out_specs=pl.BlockSpec((B, INNER), lambda b, s: (b, 0)),
            scratch_shapes=[pltpu.VMEM((2 * INNER, H), jnp.bfloat16)],
        ),
        out_shape=jax.ShapeDtypeStruct((NPAD, INNER), jnp.bfloat16),
        compiler_params=pltpu.CompilerParams(
            dimension_semantics=("arbitrary",)),
    )(bexp, x_sorted, fc1_w)


def _c2_kernel(s_ref, act_ref, fc2_ref, ws_ref, y_ref, w2bf_ref):
    b = pl.program_id(0)
    active = b < s_ref[NB]
    prev = s_ref[jnp.maximum(b - 1, 0)]
    change = jnp.logical_or(b == 0, s_ref[b] != prev)

    @pl.when(jnp.logical_and(active, change))
    def _load():
        w2bf_ref[...] = fc2_ref[0].astype(jnp.bfloat16)

    @pl.when(active)
    def _compute():
        y = jax.lax.dot_general(
            act_ref[...], w2bf_ref[...], (((1,), (1,)), ((), ())),
            preferred_element_type=jnp.float32)
        y_ref[...] = y * ws_ref[:, 0:1]


def _c2(bexp, act, fc2_w, w_sorted):
    return pl.pallas_call(
        _c2_kernel,
        grid_spec=pltpu.PrefetchScalarGridSpec(
            num_scalar_prefetch=1,
            grid=(NB,),
            in_specs=[
                pl.BlockSpec((B, INNER), lambda b, s: (b, 0)),
                pl.BlockSpec((1, H, INNER), lambda b, s: (s[b], 0, 0)),
                pl.BlockSpec((B, 128), lambda b, s: (b, 0)),
            ],
            out_specs=pl.BlockSpec((B, H), lambda b, s: (b, 0)),
            scratch_shapes=[pltpu.VMEM((H, INNER), jnp.bfloat16)],
        ),
        out_shape=jax.ShapeDtypeStruct((NPAD, H), jnp.float32),
        compiler_params=pltpu.CompilerParams(
            dimension_semantics=("arbitrary",)),
    )(bexp, act, fc2_w, w_sorted)


def _combine_kernel(y_hbm, d0_hbm, d1_hbm, out_hbm,
                    idx0_v, idx1_v, y0_v, y1_v, o_v, s0, s1, so):
    wid = lax.axis_index("s") * NC + lax.axis_index("c")
    base = wid * TPW

    pltpu.sync_copy(d0_hbm.at[wid], idx0_v)
    pltpu.sync_copy(d1_hbm.at[wid], idx1_v)

    def start_in(c):
        b = c % 2
        return (pltpu.async_copy(y_hbm.at[idx0_v.at[c]], y0_v.at[b], s0.at[b]),
                pltpu.async_copy(y_hbm.at[idx1_v.at[c]], y1_v.at[b], s1.at[b]))

    cps = {0: start_in(0), 1: start_in(1)}
    cpo = {}
    for c in range(KCH):
        b = c % 2
        for cp in cps[c]:
            cp.wait()
        if c >= 2:
            cpo[c - 2].wait()

        def body(i, carry):
            def inner(j, carry2):
                o_v[b, i, pl.ds(j * 16, 16)] = (
                    y0_v[b, i, pl.ds(j * 16, 16)]
                    + y1_v[b, i, pl.ds(j * 16, 16)])
                return carry2
            lax.fori_loop(0, H // 16, inner, 0)
            return carry
        lax.fori_loop(0, CCH, body, 0)

        cpo[c] = pltpu.async_copy(
            o_v.at[b], out_hbm.at[pl.ds(base + c * CCH, CCH)], so.at[b])
        if c + 2 < KCH:
            cps[c + 2] = start_in(c + 2)
    cpo[KCH - 2].wait()
    cpo[KCH - 1].wait()


def _combine(y_sorted, d0r, d1r):
    mesh = plsc.VectorSubcoreMesh(core_axis_name="c", subcore_axis_name="s")
    kfn = pl.kernel(
        _combine_kernel,
        mesh=mesh,
        out_type=jax.ShapeDtypeStruct((T, H), jnp.float32),
        scratch_types=[
            pltpu.VMEM((KCH, CCH), jnp.int32),
            pltpu.VMEM((KCH, CCH), jnp.int32),
            pltpu.VMEM((2, CCH, H), jnp.float32),
            pltpu.VMEM((2, CCH, H), jnp.float32),
            pltpu.VMEM((2, CCH, H), jnp.float32),
            pltpu.SemaphoreType.DMA((2,)),
            pltpu.SemaphoreType.DMA((2,)),
            pltpu.SemaphoreType.DMA((2,)),
        ],
    )
    return kfn(y_sorted, d0r, d1r)


@jax.jit
def kernel(x, gate_w, gate_b, fc1_w, fc2_w):
    dest0, dest1, w0, bexp = _routing(x, gate_w, gate_b)

    d2 = jnp.stack([dest0, dest1]).reshape(2, NW, DCH, RCH)
    x_sorted, w_sorted = _dispatch(x, d2, w0)

    act = _c1(bexp.reshape(-1), x_sorted, fc1_w)
    y = _c2(bexp.reshape(-1), act, fc2_w, w_sorted)

    d0r = dest0.reshape(NW, KCH, CCH)
    d1r = dest1.reshape(NW, KCH, CCH)
    return _combine(y, d0r, d1r)
